# Initial kernel scaffold; baseline (speedup 1.0000x reference)
import jax
import jax.numpy as jnp
from jax.experimental import pallas as pl


def kernel(hidden_states, gate_weight):
    raise NotImplementedError("write your pallas kernel here")


# speedup vs baseline: 1.7876x; 1.7876x over previous
import jax
import jax.numpy as jnp
from jax.experimental import pallas as pl
from jax.experimental.pallas import tpu as pltpu

_EPS = 1e-05
_BLK = 512


def _router_block(x_ref, w_ref, p_ref, i_ref, logits_ref):
    x = x_ref[...]
    w = w_ref[...]
    B, H = x.shape
    E = w.shape[0]
    mu = jnp.sum(x, axis=1, keepdims=True) / H
    d = x - mu
    var = jnp.sum(d * d, axis=1, keepdims=True) / H
    rstd = jax.lax.rsqrt(var + 1e-5)
    hn = d * rstd
    g = jax.lax.dot_general(
        hn.astype(jnp.bfloat16), w.astype(jnp.bfloat16),
        (((1,), (1,)), ((), ())),
        preferred_element_type=jnp.float32)
    logits = jnp.clip(g, -20.0, 20.0)
    logits_ref[...] = logits

    m = jnp.max(logits, axis=1, keepdims=True)
    e = jnp.exp(logits - m)
    p = e / jnp.sum(e, axis=1, keepdims=True)
    p = jnp.clip(p, _EPS, 1.0)

    iota = jax.lax.broadcasted_iota(jnp.int32, (B, E), 1)
    m1 = jnp.max(p, axis=1, keepdims=True)
    i1 = jnp.min(jnp.where(p == m1, iota, E), axis=1, keepdims=True)
    masked = jnp.where(iota == i1, -1.0, p)
    m2 = jnp.max(masked, axis=1, keepdims=True)
    i2 = jnp.min(jnp.where(masked == m2, iota, E), axis=1, keepdims=True)
    ps = jnp.clip(m1 + m2, _EPS, None)
    p_ref[:, 0:1] = m1 / ps
    p_ref[:, 1:2] = m2 / ps
    i_ref[:, 0:1] = i1
    i_ref[:, 1:2] = i2


def kernel(hidden_states, gate_weight):
    b, s, h = hidden_states.shape
    e = gate_weight.shape[0]
    n = b * s
    x = hidden_states.reshape(n, h)
    grid = n // _BLK
    p, idx, logits = pl.pallas_call(
        _router_block,
        grid=(grid,),
        in_specs=[
            pl.BlockSpec((_BLK, h), lambda i: (i, 0)),
            pl.BlockSpec((e, h), lambda i: (0, 0)),
        ],
        out_specs=[
            pl.BlockSpec((_BLK, 2), lambda i: (i, 0)),
            pl.BlockSpec((_BLK, 2), lambda i: (i, 0)),
            pl.BlockSpec((_BLK, e), lambda i: (i, 0)),
        ],
        out_shape=[
            jax.ShapeDtypeStruct((n, 2), jnp.float32),
            jax.ShapeDtypeStruct((n, 2), jnp.int32),
            jax.ShapeDtypeStruct((n, e), jnp.float32),
        ],
        compiler_params=pltpu.CompilerParams(
            dimension_semantics=("arbitrary",)),
    )(x, gate_weight)
    return (p, idx, logits)
